# baseline (device time: 7636 ns/iter reference)
import jax
import jax.numpy as jnp
from jax import lax
from jax.experimental import pallas as pl
from jax.experimental.pallas import tpu as pltpu

N_DEV = 4


def _combine(v_a, i_a, v_b, i_b):
    take = (v_b > v_a) | ((v_b == v_a) & (i_b < i_a))
    return jnp.where(take, v_b, v_a), jnp.where(take, i_b, i_a)


def kernel(x):
    m_per, n = x.shape

    def body(x_ref, out_ref, comm_ref, send_sems, recv_sems):
        my_pos = lax.axis_index("i")
        partner_a = my_pos ^ 1
        partner_b = 3 - my_pos

        barrier_sem = pltpu.get_barrier_semaphore()
        for p in (partner_a, partner_b):
            pl.semaphore_signal(
                barrier_sem,
                inc=1,
                device_id=(p,),
                device_id_type=pl.DeviceIdType.MESH,
            )

        xv = x_ref[:, :]
        vals = jnp.max(xv, axis=0)
        rows = lax.broadcasted_iota(jnp.int32, (m_per, n), 0)
        masked = jnp.where(xv == vals[None, :], rows, m_per)
        lidx = jnp.min(masked, axis=0)
        gidx = (lidx + my_pos * m_per).astype(jnp.float32)

        comm_ref[0, 0, :] = vals
        comm_ref[0, 1, :] = gidx

        pl.semaphore_wait(barrier_sem, 2)

        rdma_a = pltpu.make_async_remote_copy(
            src_ref=comm_ref.at[0],
            dst_ref=comm_ref.at[1],
            send_sem=send_sems.at[0],
            recv_sem=recv_sems.at[0],
            device_id=(partner_a,),
            device_id_type=pl.DeviceIdType.MESH,
        )
        rdma_a.start()
        rdma_a.wait_recv()

        v1, i1 = _combine(
            vals, gidx, comm_ref[1, 0, :], comm_ref[1, 1, :]
        )
        comm_ref[2, 0, :] = v1
        comm_ref[2, 1, :] = i1

        rdma_b = pltpu.make_async_remote_copy(
            src_ref=comm_ref.at[2],
            dst_ref=comm_ref.at[3],
            send_sem=send_sems.at[1],
            recv_sem=recv_sems.at[1],
            device_id=(partner_b,),
            device_id_type=pl.DeviceIdType.MESH,
        )
        rdma_b.start()
        rdma_b.wait_recv()

        v2, i2 = _combine(
            v1, i1, comm_ref[3, 0, :], comm_ref[3, 1, :]
        )
        out_ref[0, :] = v2
        out_ref[1, :] = i2

        rdma_a.wait_send()
        rdma_b.wait_send()

    return pl.pallas_call(
        body,
        out_shape=jax.ShapeDtypeStruct((2, n), jnp.float32),
        in_specs=[pl.BlockSpec(memory_space=pltpu.VMEM)],
        out_specs=pl.BlockSpec(memory_space=pltpu.VMEM),
        scratch_shapes=[
            pltpu.VMEM((4, 2, n), jnp.float32),
            pltpu.SemaphoreType.DMA((2,)),
            pltpu.SemaphoreType.DMA((2,)),
        ],
        compiler_params=pltpu.CompilerParams(collective_id=0),
    )(x)


# device time: 6478 ns/iter; 1.1788x vs baseline; 1.1788x over previous
import jax
import jax.numpy as jnp
from jax import lax
from jax.experimental import pallas as pl
from jax.experimental.pallas import tpu as pltpu

N_DEV = 4


def kernel(x):
    m_per, n = x.shape

    def body(x_ref, out_ref, comm_ref, send_sems, recv_sems):
        my_pos = lax.axis_index("i")

        barrier_sem = pltpu.get_barrier_semaphore()
        for k in range(1, N_DEV):
            pl.semaphore_signal(
                barrier_sem,
                inc=1,
                device_id=((my_pos + k) % N_DEV,),
                device_id_type=pl.DeviceIdType.MESH,
            )

        xv = x_ref[:, :]
        vals = jnp.max(xv, axis=0)
        rows = lax.broadcasted_iota(jnp.int32, (m_per, n), 0)
        masked = jnp.where(xv == vals[None, :], rows, m_per)
        lidx = jnp.min(masked, axis=0)
        gidx = (lidx + my_pos * m_per).astype(jnp.float32)

        comm_ref[N_DEV - 1, 0, :] = vals
        comm_ref[N_DEV - 1, 1, :] = gidx

        pl.semaphore_wait(barrier_sem, N_DEV - 1)

        rdmas = []
        for k in range(1, N_DEV):
            rdma = pltpu.make_async_remote_copy(
                src_ref=comm_ref.at[N_DEV - 1],
                dst_ref=comm_ref.at[k - 1],
                send_sem=send_sems.at[k - 1],
                recv_sem=recv_sems.at[k - 1],
                device_id=((my_pos + k) % N_DEV,),
                device_id_type=pl.DeviceIdType.MESH,
            )
            rdma.start()
            rdmas.append(rdma)

        best_v = vals
        best_i = gidx
        for k in (1, 3, 2):
            rdmas[k - 1].wait_recv()
            v = comm_ref[k - 1, 0, :]
            i = comm_ref[k - 1, 1, :]
            take = (v > best_v) | ((v == best_v) & (i < best_i))
            best_v = jnp.where(take, v, best_v)
            best_i = jnp.where(take, i, best_i)

        out_ref[0, :] = best_v
        out_ref[1, :] = best_i

        for r in rdmas:
            r.wait_send()

    return pl.pallas_call(
        body,
        out_shape=jax.ShapeDtypeStruct((2, n), jnp.float32),
        in_specs=[pl.BlockSpec(memory_space=pltpu.VMEM)],
        out_specs=pl.BlockSpec(memory_space=pltpu.VMEM),
        scratch_shapes=[
            pltpu.VMEM((N_DEV, 2, n), jnp.float32),
            pltpu.SemaphoreType.DMA((N_DEV - 1,)),
            pltpu.SemaphoreType.DMA((N_DEV - 1,)),
        ],
        compiler_params=pltpu.CompilerParams(collective_id=0),
    )(x)


# device time: 6450 ns/iter; 1.1839x vs baseline; 1.0043x over previous
import jax
import jax.numpy as jnp
from jax import lax
from jax.experimental import pallas as pl
from jax.experimental.pallas import tpu as pltpu

N_DEV = 4


def kernel(x):
    m_per, n = x.shape

    def body(x_ref, out_ref, comm_ref, send_sems, recv_sems):
        my_pos = lax.axis_index("i")

        barrier_sem = pltpu.get_barrier_semaphore()
        for k in range(1, N_DEV):
            pl.semaphore_signal(
                barrier_sem,
                inc=1,
                device_id=((my_pos + k) % N_DEV,),
                device_id_type=pl.DeviceIdType.MESH,
            )

        xv = x_ref[:, :]
        vals = jnp.max(xv, axis=0)
        lidx = jnp.argmax(xv, axis=0)
        gidx = (lidx + my_pos * m_per).astype(jnp.float32)

        comm_ref[N_DEV - 1, 0, :] = vals
        comm_ref[N_DEV - 1, 1, :] = gidx

        pl.semaphore_wait(barrier_sem, N_DEV - 1)

        rdmas = []
        for k in range(1, N_DEV):
            rdma = pltpu.make_async_remote_copy(
                src_ref=comm_ref.at[N_DEV - 1],
                dst_ref=comm_ref.at[k - 1],
                send_sem=send_sems.at[k - 1],
                recv_sem=recv_sems.at[k - 1],
                device_id=((my_pos + k) % N_DEV,),
                device_id_type=pl.DeviceIdType.MESH,
            )
            rdma.start()
            rdmas.append(rdma)

        best_v = vals
        best_i = gidx
        for k in (1, 3, 2):
            rdmas[k - 1].wait_recv()
            v = comm_ref[k - 1, 0, :]
            i = comm_ref[k - 1, 1, :]
            take = (v > best_v) | ((v == best_v) & (i < best_i))
            best_v = jnp.where(take, v, best_v)
            best_i = jnp.where(take, i, best_i)

        out_ref[0, :] = best_v
        out_ref[1, :] = best_i

        for r in rdmas:
            r.wait_send()

    return pl.pallas_call(
        body,
        out_shape=jax.ShapeDtypeStruct((2, n), jnp.float32),
        in_specs=[pl.BlockSpec(memory_space=pltpu.VMEM)],
        out_specs=pl.BlockSpec(memory_space=pltpu.VMEM),
        scratch_shapes=[
            pltpu.VMEM((N_DEV, 2, n), jnp.float32),
            pltpu.SemaphoreType.DMA((N_DEV - 1,)),
            pltpu.SemaphoreType.DMA((N_DEV - 1,)),
        ],
        compiler_params=pltpu.CompilerParams(collective_id=0),
    )(x)
